# Initial kernel scaffold; baseline (speedup 1.0000x reference)
import jax
import jax.numpy as jnp
from jax.experimental import pallas as pl


def kernel(node_feats, edge_feats, chi_scalar, cutoffs, senders, receivers, W_rad1, b_rad1, W_rad2, b_rad2, W_sph1, b_sph1, W_sph2, b_sph2, Wq, Wk, Wv):
    raise NotImplementedError("write your pallas kernel here")


# speedup vs baseline: 26.8096x; 26.8096x over previous
import functools
import jax
import jax.numpy as jnp
from jax import lax
from jax.experimental import pallas as pl
from jax.experimental.pallas import tpu as pltpu
from jax.experimental.pallas import tpu_sc as plsc

N = 10000
E = 320000
D = 128
H = 8
HD = 16

NC = 2
NS = 16
NW = NC * NS
BE = 64
NBLK = E // BE
ZC = 64
NZFULL = N // ZC
ZREM = N - NZFULL * ZC
SCALE = 1.0 / (4.0 * 32.0)


def _edge_mlp_body(ef, chi, cut, w1, b1, w2, b2, w3, b3, w4, b4, out):
    h1 = jnp.dot(ef[...], w1[...], preferred_element_type=jnp.float32) + b1[...]
    h1 = h1 * jax.nn.sigmoid(h1)
    r = jnp.dot(h1, w2[...], preferred_element_type=jnp.float32) + b2[...]
    h2 = jnp.dot(chi[...], w3[...], preferred_element_type=jnp.float32) + b3[...]
    h2 = h2 * jax.nn.sigmoid(h2)
    w = r + jnp.dot(h2, w4[...], preferred_element_type=jnp.float32) + b4[...]
    out[...] = w * (cut[...] * SCALE)


def _edge_mlp(ef, chi, cut, w1, b1, w2, b2, w3, b3, w4, b4):
    BEW = 1600
    grid = E // BEW
    full = lambda shape: pl.BlockSpec(shape, lambda i: (0, 0))
    return pl.pallas_call(
        _edge_mlp_body,
        grid=(grid,),
        in_specs=[
            pl.BlockSpec((BEW, 16), lambda i: (i, 0)),
            pl.BlockSpec((BEW, 16), lambda i: (i, 0)),
            pl.BlockSpec((BEW, 1), lambda i: (i, 0)),
            full((16, 64)), full((1, 64)),
            full((64, D)), full((1, D)),
            full((16, 64)), full((1, 64)),
            full((64, D)), full((1, D)),
        ],
        out_specs=pl.BlockSpec((BEW, D), lambda i: (i, 0)),
        out_shape=jax.ShapeDtypeStruct((E, D), jnp.float32),
    )(ef, chi, cut, w1, b1, w2, b2, w3, b3, w4, b4)


def _qkv_body(nf, wq, wk, wv, qo, ko, vo):
    x = nf[...]
    qo[...] = jnp.dot(x, wq[...], preferred_element_type=jnp.float32)
    ko[...] = jnp.dot(x, wk[...], preferred_element_type=jnp.float32)
    vo[...] = jnp.dot(x, wv[...], preferred_element_type=jnp.float32)


def _qkv(nf, wq, wk, wv):
    BN = 1000
    grid = N // BN
    full = lambda: pl.BlockSpec((D, D), lambda i: (0, 0))
    s = jax.ShapeDtypeStruct((N, D), jnp.float32)
    return pl.pallas_call(
        _qkv_body,
        grid=(grid,),
        in_specs=[pl.BlockSpec((BN, D), lambda i: (i, 0)), full(), full(), full()],
        out_specs=[pl.BlockSpec((BN, D), lambda i: (i, 0))] * 3,
        out_shape=[s, s, s],
    )(nf, wq, wk, wv)


def _sc_body(wp, qt, kt, vt, snd, rcv, out,
             sidx_v, ridx_v, w_v, q_v, k_v, v_v, c_v, acc,
             sem1, sem2, sem3):
    c = lax.axis_index("c")
    s = lax.axis_index("s")
    wid = c * NS + s

    def zrow(r, carry):
        for j in range(D // 16):
            c_v[r, pl.ds(j * 16, 16)] = jnp.zeros((16,), jnp.float32)
        return carry
    lax.fori_loop(0, ZC, zrow, 0)

    def zchunk(j, carry):
        i = s + j * NS
        @pl.when(i < NZFULL)
        def _():
            pltpu.sync_copy(c_v, acc.at[pl.ds(i * ZC, ZC)])
        return carry
    lax.fori_loop(0, (NZFULL + NS - 1) // NS, zchunk, 0)

    @pl.when(s == 0)
    def _():
        pltpu.sync_copy(c_v.at[pl.ds(0, ZREM)], acc.at[pl.ds(NZFULL * ZC, ZREM)])
    plsc.subcore_barrier()

    def block(j, carry):
        b = wid + j * NW
        @pl.when(b < NBLK)
        def _():
            base = b * BE
            pltpu.sync_copy(snd.at[pl.ds(base, BE)], sidx_v)
            pltpu.sync_copy(rcv.at[pl.ds(base, BE)], ridx_v)
            cp_w = pltpu.async_copy(wp.at[pl.ds(base, BE)], w_v, sem1)
            cp_q = pltpu.async_copy(qt.at[ridx_v], q_v, sem2)
            cp_k = pltpu.async_copy(kt.at[sidx_v], k_v, sem3)
            cp_w.wait()
            cp_q.wait()
            cp_k.wait()
            cp_v = pltpu.async_copy(vt.at[sidx_v], v_v, sem1)
            cp_v.wait()

            lanes = lax.iota(jnp.int32, 16)
            perms = [lanes ^ jnp.int32(1 << jj) for jj in (3, 2, 1, 0)]
            dnums = lax.GatherDimensionNumbers(
                offset_dims=(), collapsed_slice_dims=(0,), start_index_map=(0,))

            def lane_perm(x, pm):
                return lax.gather(x, pm[:, None], dnums, slice_sizes=(1,),
                                  mode=lax.GatherScatterMode.PROMISE_IN_BOUNDS)

            def edge(e, carry2):
                for h in range(H):
                    dsl = pl.ds(h * HD, HD)
                    p = q_v[e, dsl] * w_v[e, dsl] * k_v[e, dsl]
                    for pm in perms:
                        p = p + lane_perm(p, pm)
                    c_v[e, dsl] = p * v_v[e, dsl]
                return carry2
            lax.fori_loop(0, BE, edge, 0)

            pltpu.sync_copy(c_v, acc.at[ridx_v], add=True)
        return carry
    lax.fori_loop(0, (NBLK + NW - 1) // NW, block, 0)

    plsc.subcore_barrier()

    def ochunk(j, carry):
        i = s + j * NS
        @pl.when(i < NZFULL)
        def _():
            r0 = i * ZC
            pltpu.sync_copy(acc.at[pl.ds(r0, ZC)], c_v)
            pltpu.sync_copy(c_v, out.at[c, pl.ds(r0, ZC)])
        return carry
    lax.fori_loop(0, (NZFULL + NS - 1) // NS, ochunk, 0)

    @pl.when(s == 0)
    def _():
        r0 = NZFULL * ZC
        pltpu.sync_copy(acc.at[pl.ds(r0, ZREM)], c_v.at[pl.ds(0, ZREM)])
        pltpu.sync_copy(c_v.at[pl.ds(0, ZREM)], out.at[c, pl.ds(r0, ZREM)])


def _sc_scatter(wp, qt, kt, vt, snd, rcv):
    mesh = plsc.VectorSubcoreMesh(core_axis_name="c", subcore_axis_name="s")
    f = functools.partial(
        pl.kernel,
        out_type=jax.ShapeDtypeStruct((NC, N, D), jnp.float32),
        mesh=mesh,
        scratch_types=[
            pltpu.VMEM((BE,), jnp.int32),
            pltpu.VMEM((BE,), jnp.int32),
            pltpu.VMEM((BE, D), jnp.float32),
            pltpu.VMEM((BE, D), jnp.float32),
            pltpu.VMEM((BE, D), jnp.float32),
            pltpu.VMEM((BE, D), jnp.float32),
            pltpu.VMEM((BE, D), jnp.float32),
            pltpu.VMEM_SHARED((N, D), jnp.float32),
            pltpu.SemaphoreType.DMA,
            pltpu.SemaphoreType.DMA,
            pltpu.SemaphoreType.DMA,
        ],
    )(_sc_body)
    return f(wp, qt, kt, vt, snd, rcv)


def _sum_body(p, o):
    o[...] = p[0] + p[1]


def _sum_partials(parts):
    BN = 1000
    return pl.pallas_call(
        _sum_body,
        grid=(N // BN,),
        in_specs=[pl.BlockSpec((NC, BN, D), lambda i: (0, i, 0))],
        out_specs=pl.BlockSpec((BN, D), lambda i: (i, 0)),
        out_shape=jax.ShapeDtypeStruct((N, D), jnp.float32),
    )(parts)


def kernel(node_feats, edge_feats, chi_scalar, cutoffs, senders, receivers,
           W_rad1, b_rad1, W_rad2, b_rad2,
           W_sph1, b_sph1, W_sph2, b_sph2,
           Wq, Wk, Wv):
    wp = _edge_mlp(edge_feats, chi_scalar,
                   cutoffs.astype(jnp.float32).reshape(E, 1),
                   W_rad1, b_rad1.reshape(1, 64), W_rad2, b_rad2.reshape(1, D),
                   W_sph1, b_sph1.reshape(1, 64), W_sph2, b_sph2.reshape(1, D))
    qt, kt, vt = _qkv(node_feats, Wq, Wk, Wv)
    parts = _sc_scatter(wp, qt, kt, vt,
                        senders.astype(jnp.int32), receivers.astype(jnp.int32))
    return _sum_partials(parts)
